# baseline (speedup 1.0000x reference)
import jax
import jax.numpy as jnp
from jax.experimental import pallas as pl
from jax.experimental.pallas import tpu as pltpu


def _copy_body(f_in, f_out, mask_out):
    f_out[...] = f_in[...]
    mask_out[...] = jnp.ones(mask_out.shape, dtype=jnp.bool_)


def kernel(features, means, xy_coords, A):
    B, V, G, C = features.shape
    del xy_coords, A
    BV = B * V
    rows = BV * G
    f2 = features.reshape(rows, C)

    ROWS = 8192
    n_prog = rows // ROWS

    f_out, mask = pl.pallas_call(
        _copy_body,
        grid=(n_prog,),
        in_specs=[pl.BlockSpec((ROWS, C), lambda i: (i, 0))],
        out_specs=[
            pl.BlockSpec((ROWS, C), lambda i: (i, 0)),
            pl.BlockSpec((BV, G), lambda i: (0, 0)),
        ],
        out_shape=[
            jax.ShapeDtypeStruct((rows, C), features.dtype),
            jax.ShapeDtypeStruct((BV, G), jnp.bool_),
        ],
    )(f2)

    return (
        f_out.reshape(B, V * G, C),
        means.reshape(B, V * G, 3),
        mask.reshape(B, V, G),
    )


# speedup vs baseline: 1.0666x; 1.0666x over previous
import jax
import jax.numpy as jnp
from jax.experimental import pallas as pl
from jax.experimental.pallas import tpu as pltpu


def _copy_body(f_in, f_out, mask_out):
    f_out[...] = f_in[...]
    mask_out[...] = jnp.ones(mask_out.shape, dtype=jnp.bool_)


def kernel(features, means, xy_coords, A):
    B, V, G, C = features.shape
    del xy_coords, A
    BV = B * V
    rows = BV * G
    f2 = features.reshape(rows, C)

    ROWS = 16384
    n_prog = rows // ROWS

    f_out, mask = pl.pallas_call(
        _copy_body,
        grid=(n_prog,),
        in_specs=[pl.BlockSpec((ROWS, C), lambda i: (i, 0))],
        out_specs=[
            pl.BlockSpec((ROWS, C), lambda i: (i, 0)),
            pl.BlockSpec((BV, G), lambda i: (0, 0)),
        ],
        out_shape=[
            jax.ShapeDtypeStruct((rows, C), features.dtype),
            jax.ShapeDtypeStruct((BV, G), jnp.bool_),
        ],
    )(f2)

    return (
        f_out.reshape(B, V * G, C),
        means.reshape(B, V * G, 3),
        mask.reshape(B, V, G),
    )
